# baseline (speedup 1.0000x reference)
import functools
import math

import jax
import jax.numpy as jnp
from jax.experimental import pallas as pl
from jax.experimental.pallas import tpu as pltpu

BLOCK = 32
MIN_K, MAX_K, ALPHA, THR = 4, 16, 0.1, 0.5
SUB = 4
ROWS = SUB * BLOCK
GB = 128
INV_SQRT_D = 1.0 / math.sqrt(64.0)
NOB = float(ROWS - BLOCK)


def _masks_tall(nrows):
    i = jax.lax.broadcasted_iota(jnp.int32, (nrows, ROWS), 0)
    j = jax.lax.broadcasted_iota(jnp.int32, (nrows, ROWS), 1)
    blk = ((i // BLOCK) % SUB) == (j // BLOCK)
    tril = (i % BLOCK) >= (j % BLOCK)
    decay = 1.0 - ALPHA * jnp.abs(i % BLOCK - j % BLOCK).astype(jnp.float32) / BLOCK
    dscale = jnp.where(blk & tril, decay * INV_SQRT_D, 0.0)
    return blk, dscale


def _scores_all(q, k):
    nt = q.shape[0] // ROWS
    outs = []
    for t in range(nt):
        rows = slice(t * ROWS, (t + 1) * ROWS)
        outs.append(jax.lax.dot_general(q[rows], k[rows], (((1,), (1,)), ((), ())),
                                        preferred_element_type=jnp.float32))
    return jnp.concatenate(outs, axis=0)


def _f_em_denom(s, dscale):
    f = jnp.maximum(s, 0.0) * dscale
    m = jnp.max(f, axis=-1, keepdims=True)
    eb = jnp.exp(f - m)
    denom = jnp.sum(eb, axis=-1, keepdims=True) - NOB * jnp.exp(-m)
    return f, eb, denom


def _extract_compact(a, nrows):
    rb = (jax.lax.broadcasted_iota(jnp.int32, (nrows, BLOCK), 0) // BLOCK) % SUB
    out = jnp.zeros((nrows, BLOCK), jnp.float32)
    for u in range(SUB):
        out = out + jnp.where(rb == u, a[:, u * BLOCK:(u + 1) * BLOCK], 0.0)
    return out


def _ent_kernel(x_ref, wqT_ref, wkT_ref, out_ref):
    nr = GB * BLOCK
    X = x_ref[...].reshape(nr, x_ref.shape[-1])
    q = jnp.dot(X, wqT_ref[...], preferred_element_type=jnp.float32)
    k = jnp.dot(X, wkT_ref[...], preferred_element_type=jnp.float32)
    _, dscale = _masks_tall(nr)
    s = _scores_all(q, k)
    _, eb, denom = _f_em_denom(s, dscale)
    pc = _extract_compact(eb, nr) / denom
    acc = -jnp.sum(pc * jnp.log(pc + 1e-9))
    out_ref[...] = acc.reshape(1, 1, 1)


def _hybrid_kernel(x_ref, wqT_ref, wkT_ref, wvT_ref, p1T_ref, p2T_ref, out_ref):
    nr = GB * BLOCK
    X = x_ref[...].reshape(nr, x_ref.shape[-1])
    q = jnp.dot(X, wqT_ref[...], preferred_element_type=jnp.float32)
    k = jnp.dot(X, wkT_ref[...], preferred_element_type=jnp.float32)
    v = jnp.dot(X, wvT_ref[...], preferred_element_type=jnp.float32)
    blk, dscale = _masks_tall(nr)
    s = _scores_all(q, k)
    _, eb, denom = _f_em_denom(s, dscale)
    pc = _extract_compact(eb, nr) / denom
    lat = jnp.maximum(jnp.dot(pc, p1T_ref[...], preferred_element_type=jnp.float32), 0.0)
    lg = jnp.dot(lat, p2T_ref[...], preferred_element_type=jnp.float32)
    m2 = jnp.max(lg, axis=-1, keepdims=True)
    e2 = jnp.exp(lg - m2)
    ac = e2 / jnp.sum(e2, axis=-1, keepdims=True)
    abd = jnp.where(blk, jnp.concatenate([ac] * SUB, axis=1), 0.0)
    for t in range(GB // SUB):
        rows = slice(t * ROWS, (t + 1) * ROWS)
        o = jnp.dot(abd[rows], v[rows], preferred_element_type=jnp.float32)
        out_ref[t * SUB:(t + 1) * SUB] = o.reshape(SUB, BLOCK, 64)


GB_S = 16


def _sparse_kernel(x_ref, wqT_ref, wkT_ref, wvT_ref, sc_ref, out_ref):
    kk = sc_ref[0, 0]
    amr = sc_ref[0, 1]
    gamma = sc_ref[0, 2]
    nr = GB_S * BLOCK
    X = x_ref[...].reshape(nr, x_ref.shape[-1])
    q = jnp.dot(X, wqT_ref[...], preferred_element_type=jnp.float32)
    k = jnp.dot(X, wkT_ref[...], preferred_element_type=jnp.float32)
    v = jnp.dot(X, wvT_ref[...], preferred_element_type=jnp.float32)
    blk, dscale = _masks_tall(nr)
    s = _scores_all(q, k)
    f = jnp.maximum(s, 0.0) * dscale
    fc = _extract_compact(f, nr)
    mean = jnp.mean(fc, axis=-1, keepdims=True)
    var = jnp.sum((fc - mean) ** 2, axis=-1, keepdims=True) / (BLOCK - 1)
    sigma = jnp.sqrt(var)
    m = jnp.max(fc, axis=-1, keepdims=True)
    denom = jnp.maximum(m, sigma) + 1e-6
    nw = jnp.clip(jnp.floor(amr * fc / denom), 0.0, amr)
    col = jax.lax.broadcasted_iota(jnp.int32, (nr, BLOCK), 1)
    rank = jnp.zeros((nr, BLOCK), jnp.float32)
    for sft in range(1, BLOCK):
        fs = jnp.concatenate([fc[:, sft:], fc[:, :sft]], axis=1)
        cond = (fs > fc) | ((fs == fc) & (col + sft >= BLOCK))
        rank = rank + cond.astype(jnp.float32)
    sel = rank < jnp.minimum(kk, float(MAX_K))
    w = jnp.where(sel, nw, 0.0) / gamma
    wb = jnp.where(blk, jnp.concatenate([w] * SUB, axis=1), 0.0)
    for t in range(GB_S // SUB):
        rows = slice(t * ROWS, (t + 1) * ROWS)
        o = jnp.dot(wb[rows], v[rows], preferred_element_type=jnp.float32)
        out_ref[t * SUB:(t + 1) * SUB] = o.reshape(SUB, BLOCK, 64)


def kernel(x, Wk, Wq, Wv, P1, P2, gamma):
    B, T, C = x.shape
    nsteps = B // GB
    wqT = Wq.T
    wkT = Wk.T
    wvT = Wv.T
    p1T = P1.T
    p2T = P2.T

    w_spec = pl.BlockSpec((C, 64), lambda i: (0, 0))
    p_spec = pl.BlockSpec((T, T), lambda i: (0, 0))
    x_spec = pl.BlockSpec((GB, T, C), lambda i: (i, 0, 0))
    out_spec = pl.BlockSpec((GB, T, 64), lambda i: (i, 0, 0))

    ent = pl.pallas_call(
        _ent_kernel,
        grid=(nsteps,),
        in_specs=[x_spec, w_spec, w_spec],
        out_specs=pl.BlockSpec((1, 1, 1), lambda i: (i, 0, 0)),
        out_shape=jax.ShapeDtypeStruct((nsteps, 1, 1), jnp.float32),
    )(x, wqT, wkT)

    a = jnp.sum(ent) / (B * T * math.log(T))
    hybrid = a > THR
    kk = jnp.clip(jnp.floor(MIN_K + (MAX_K - MIN_K) * a), MIN_K, MAX_K)
    amr = jnp.floor(15 + (127 - 15) * a)

    def _hybrid_branch(ops):
        x, wqT, wkT, wvT, p1T, p2T, _ = ops
        return pl.pallas_call(
            _hybrid_kernel,
            grid=(nsteps,),
            in_specs=[x_spec, w_spec, w_spec, w_spec, p_spec, p_spec],
            out_specs=out_spec,
            out_shape=jax.ShapeDtypeStruct((B, T, 64), jnp.float32),
        )(x, wqT, wkT, wvT, p1T, p2T)

    def _sparse_branch(ops):
        x, wqT, wkT, wvT, _, _, sc = ops
        return pl.pallas_call(
            _sparse_kernel,
            grid=(B // GB_S,),
            in_specs=[pl.BlockSpec((GB_S, T, C), lambda i: (i, 0, 0)),
                      w_spec, w_spec, w_spec,
                      pl.BlockSpec(memory_space=pltpu.SMEM)],
            out_specs=pl.BlockSpec((GB_S, T, 64), lambda i: (i, 0, 0)),
            out_shape=jax.ShapeDtypeStruct((B, T, 64), jnp.float32),
        )(x, wqT, wkT, wvT, sc)

    sc = jnp.stack([kk, amr, gamma.astype(jnp.float32)]).reshape(1, 3)
    ops = (x, wqT, wkT, wvT, p1T, p2T, sc)
    return jax.lax.cond(hybrid, _hybrid_branch, _sparse_branch, ops)


# speedup vs baseline: 1.2394x; 1.2394x over previous
import functools
import math

import jax
import jax.numpy as jnp
from jax.experimental import pallas as pl
from jax.experimental.pallas import tpu as pltpu

BLOCK = 32
MIN_K, MAX_K, ALPHA, THR = 4, 16, 0.1, 0.5
SUB = 4
ROWS = SUB * BLOCK
GB = 64
GB_S = 16
INV_SQRT_D = 1.0 / math.sqrt(64.0)
NOB = float(ROWS - BLOCK)

_DNT = (((1,), (1,)), ((), ()))


def _masks_tall(nrows):
    i = jax.lax.broadcasted_iota(jnp.int32, (nrows, ROWS), 0)
    j = jax.lax.broadcasted_iota(jnp.int32, (nrows, ROWS), 1)
    blk = ((i // BLOCK) % SUB) == (j // BLOCK)
    tril = (i % BLOCK) >= (j % BLOCK)
    decay = 1.0 - ALPHA * jnp.abs(i % BLOCK - j % BLOCK).astype(jnp.float32) / BLOCK
    dscale = jnp.where(blk & tril, decay * INV_SQRT_D, 0.0)
    return blk, dscale


def _scores_all(q, k):
    nt = q.shape[0] // ROWS
    outs = []
    for t in range(nt):
        rows = slice(t * ROWS, (t + 1) * ROWS)
        outs.append(jax.lax.dot_general(q[rows], k[rows], _DNT,
                                        preferred_element_type=jnp.float32))
    return jnp.concatenate(outs, axis=0)


def _probs_corr(s, dscale):
    f = jnp.maximum(s, 0.0) * dscale
    m = jnp.max(f, axis=-1, keepdims=True)
    eb = jnp.exp(f - m)
    em = jnp.exp(-m)
    denom = jnp.sum(eb, axis=-1, keepdims=True) - NOB * em
    p = eb / denom
    return f, p, em / denom


def _ent_kernel(x_ref, wq_ref, wk_ref, out_ref, acc_ref):
    step = pl.program_id(0)
    nsteps = pl.num_programs(0)
    nr = GB * BLOCK
    X = x_ref[...].reshape(nr, x_ref.shape[-1])
    q = jax.lax.dot_general(X, wq_ref[...], _DNT, preferred_element_type=jnp.float32)
    k = jax.lax.dot_general(X, wk_ref[...], _DNT, preferred_element_type=jnp.float32)
    _, dscale = _masks_tall(nr)
    s = _scores_all(q, k)
    _, p, po = _probs_corr(s, dscale)
    full = jnp.sum(p * jnp.log(p + 1e-9))
    corr = NOB * jnp.sum(po * jnp.log(po + 1e-9))
    ent = corr - full

    @pl.when(step == 0)
    def _init():
        acc_ref[0, 0] = 0.0

    acc_ref[0, 0] += ent

    @pl.when(step == nsteps - 1)
    def _fin():
        total_rows = float(nsteps * GB * BLOCK)
        a = acc_ref[0, 0] / (total_rows * math.log(BLOCK))
        out_ref[0, 0] = jnp.where(a > THR, 1.0, 0.0)
        out_ref[0, 1] = jnp.clip(jnp.floor(MIN_K + (MAX_K - MIN_K) * a),
                                 float(MIN_K), float(MAX_K))
        out_ref[0, 2] = jnp.floor(15.0 + (127.0 - 15.0) * a)


def _hybrid_kernel(x_ref, wq_ref, wk_ref, wv_ref, p1b_ref, p2b_ref, out_ref):
    nr = GB * BLOCK
    X = x_ref[...].reshape(nr, x_ref.shape[-1])
    q = jax.lax.dot_general(X, wq_ref[...], _DNT, preferred_element_type=jnp.float32)
    k = jax.lax.dot_general(X, wk_ref[...], _DNT, preferred_element_type=jnp.float32)
    v = jax.lax.dot_general(X, wv_ref[...], _DNT, preferred_element_type=jnp.float32)
    blk, dscale = _masks_tall(nr)
    s = _scores_all(q, k)
    _, p, _ = _probs_corr(s, dscale)
    lat = jnp.maximum(jnp.dot(p, p1b_ref[...], preferred_element_type=jnp.float32), 0.0)
    lg = jnp.dot(lat, p2b_ref[...], preferred_element_type=jnp.float32)
    lm = jnp.max(lg, axis=-1, keepdims=True)
    e2 = jnp.where(blk, jnp.exp(lg - lm), 0.0)
    a = e2 / jnp.sum(e2, axis=-1, keepdims=True)
    for t in range(GB // SUB):
        rows = slice(t * ROWS, (t + 1) * ROWS)
        o = jnp.dot(a[rows], v[rows], preferred_element_type=jnp.float32)
        out_ref[t * SUB:(t + 1) * SUB] = o.reshape(SUB, BLOCK, 64)


def _sparse_kernel(x_ref, wq_ref, wk_ref, wv_ref, sc_ref, g_ref, out_ref):
    kk = sc_ref[0, 1]
    amr = sc_ref[0, 2]
    gamma = g_ref[0, 0]
    nr = GB_S * BLOCK
    X = x_ref[...].reshape(nr, x_ref.shape[-1])
    q = jax.lax.dot_general(X, wq_ref[...], _DNT, preferred_element_type=jnp.float32)
    k = jax.lax.dot_general(X, wk_ref[...], _DNT, preferred_element_type=jnp.float32)
    v = jax.lax.dot_general(X, wv_ref[...], _DNT, preferred_element_type=jnp.float32)
    blk, dscale = _masks_tall(nr)
    s = _scores_all(q, k)
    f = jnp.maximum(s, 0.0) * dscale
    rb = (jax.lax.broadcasted_iota(jnp.int32, (nr, BLOCK), 0) // BLOCK) % SUB
    fc = jnp.zeros((nr, BLOCK), jnp.float32)
    for u in range(SUB):
        fc = fc + jnp.where(rb == u, f[:, u * BLOCK:(u + 1) * BLOCK], 0.0)
    mean = jnp.mean(fc, axis=-1, keepdims=True)
    var = jnp.sum((fc - mean) ** 2, axis=-1, keepdims=True) / (BLOCK - 1)
    sigma = jnp.sqrt(var)
    m = jnp.max(fc, axis=-1, keepdims=True)
    denom = jnp.maximum(m, sigma) + 1e-6
    nw = jnp.clip(jnp.floor(amr * fc / denom), 0.0, amr)
    col = jax.lax.broadcasted_iota(jnp.int32, (nr, BLOCK), 1)
    rank = jnp.zeros((nr, BLOCK), jnp.float32)
    for sft in range(1, BLOCK):
        fs = jnp.concatenate([fc[:, sft:], fc[:, :sft]], axis=1)
        cond = (fs > fc) | ((fs == fc) & (col + sft >= BLOCK))
        rank = rank + cond.astype(jnp.float32)
    sel = rank < jnp.minimum(kk, float(MAX_K))
    w = jnp.where(sel, nw, 0.0) / gamma
    wb = jnp.where(blk, jnp.concatenate([w] * SUB, axis=1), 0.0)
    for t in range(GB_S // SUB):
        rows = slice(t * ROWS, (t + 1) * ROWS)
        o = jnp.dot(wb[rows], v[rows], preferred_element_type=jnp.float32)
        out_ref[t * SUB:(t + 1) * SUB] = o.reshape(SUB, BLOCK, 64)


def kernel(x, Wk, Wq, Wv, P1, P2, gamma):
    B, T, C = x.shape
    nsteps = B // GB
    eye = jnp.eye(SUB, dtype=jnp.float32)
    p1b = jnp.kron(eye, P1.T)
    p2b = jnp.kron(eye, P2.T)

    w_spec = pl.BlockSpec((64, C), lambda i: (0, 0))
    x_spec = pl.BlockSpec((GB, T, C), lambda i: (i, 0, 0))
    out_spec = pl.BlockSpec((GB, T, 64), lambda i: (i, 0, 0))

    sc = pl.pallas_call(
        _ent_kernel,
        grid=(nsteps,),
        in_specs=[x_spec, w_spec, w_spec],
        out_specs=pl.BlockSpec(memory_space=pltpu.SMEM),
        out_shape=jax.ShapeDtypeStruct((1, 3), jnp.float32),
        scratch_shapes=[pltpu.SMEM((1, 1), jnp.float32)],
    )(x, Wq, Wk)

    hybrid = sc[0, 0] > 0.5

    def _hybrid_branch(ops):
        x, Wq, Wk, Wv, p1b, p2b, _, _ = ops
        return pl.pallas_call(
            _hybrid_kernel,
            grid=(nsteps,),
            in_specs=[x_spec, w_spec, w_spec, w_spec,
                      pl.BlockSpec((ROWS, ROWS), lambda i: (0, 0)),
                      pl.BlockSpec((ROWS, ROWS), lambda i: (0, 0))],
            out_specs=out_spec,
            out_shape=jax.ShapeDtypeStruct((B, T, 64), jnp.float32),
        )(x, Wq, Wk, Wv, p1b, p2b)

    def _sparse_branch(ops):
        x, Wq, Wk, Wv, _, _, sc, g = ops
        return pl.pallas_call(
            _sparse_kernel,
            grid=(B // GB_S,),
            in_specs=[pl.BlockSpec((GB_S, T, C), lambda i: (i, 0, 0)),
                      w_spec, w_spec, w_spec,
                      pl.BlockSpec(memory_space=pltpu.SMEM),
                      pl.BlockSpec(memory_space=pltpu.SMEM)],
            out_specs=pl.BlockSpec((GB_S, T, 64), lambda i: (i, 0, 0)),
            out_shape=jax.ShapeDtypeStruct((B, T, 64), jnp.float32),
        )(x, Wq, Wk, Wv, sc, g)

    g = gamma.astype(jnp.float32).reshape(1, 1)
    ops = (x, Wq, Wk, Wv, p1b, p2b, sc, g)
    return jax.lax.cond(hybrid, _hybrid_branch, _sparse_branch, ops)


# speedup vs baseline: 2.0001x; 1.6137x over previous
import functools
import math

import jax
import jax.numpy as jnp
from jax.experimental import pallas as pl
from jax.experimental.pallas import tpu as pltpu

BLOCK = 32
MIN_K, MAX_K, ALPHA, THR = 4, 16, 0.1, 0.5
SUB = 4
ROWS = SUB * BLOCK
GB = 64
GB_S = 16
INV_SQRT_D = 1.0 / math.sqrt(64.0)
NOB = float(ROWS - BLOCK)

_DNT = (((1,), (1,)), ((), ()))
_DNN = (((1,), (0,)), ((), ()))


def _masks_tall(nrows):
    i = jax.lax.broadcasted_iota(jnp.int32, (nrows, ROWS), 0)
    j = jax.lax.broadcasted_iota(jnp.int32, (nrows, ROWS), 1)
    blk = ((i // BLOCK) % SUB) == (j // BLOCK)
    tril = (i % BLOCK) >= (j % BLOCK)
    decay = 1.0 - ALPHA * jnp.abs(i % BLOCK - j % BLOCK).astype(jnp.float32) / BLOCK
    dscale = jnp.where(blk & tril, decay * INV_SQRT_D, 0.0)
    return blk, dscale


def _scores_all(y, xt):
    nt = y.shape[0] // ROWS
    outs = []
    for t in range(nt):
        rows = slice(t * ROWS, (t + 1) * ROWS)
        outs.append(jax.lax.dot_general(y[rows], xt[rows], _DNT,
                                        preferred_element_type=jnp.float32))
    return jnp.concatenate(outs, axis=0)


def _probs_corr(s, dscale):
    f = jnp.maximum(s, 0.0) * dscale
    m = jnp.max(f, axis=-1, keepdims=True)
    eb = jnp.exp(f - m)
    em = jnp.exp(-m)
    denom = jnp.sum(eb, axis=-1, keepdims=True) - NOB * em
    p = eb / denom
    return f, p, em / denom


def _fused_kernel(x_ref, m_ref, wv_ref, p1b_ref, p2b_ref,
                  out_ref, sc_ref, acc_ref):
    step = pl.program_id(0)
    nsteps = pl.num_programs(0)
    nr = GB * BLOCK
    X = x_ref[...].reshape(nr, x_ref.shape[-1])
    y = jax.lax.dot_general(X, m_ref[...], _DNN, preferred_element_type=jnp.float32)
    v = jax.lax.dot_general(X, wv_ref[...], _DNT, preferred_element_type=jnp.float32)
    blk, dscale = _masks_tall(nr)
    s = _scores_all(y, X)
    _, p, po = _probs_corr(s, dscale)

    full = jnp.sum(p * jnp.log(p + 1e-9))
    corr = NOB * jnp.sum(po * jnp.log(po + 1e-9))

    @pl.when(step == 0)
    def _init():
        acc_ref[0, 0] = 0.0

    acc_ref[0, 0] += corr - full

    @pl.when(step == nsteps - 1)
    def _fin():
        total_rows = float(nsteps * GB * BLOCK)
        a = acc_ref[0, 0] / (total_rows * math.log(BLOCK))
        sc_ref[0, 0] = jnp.where(a > THR, 1.0, 0.0)
        sc_ref[0, 1] = jnp.clip(jnp.floor(MIN_K + (MAX_K - MIN_K) * a),
                                float(MIN_K), float(MAX_K))
        sc_ref[0, 2] = jnp.floor(15.0 + (127.0 - 15.0) * a)

    lat = jnp.maximum(jnp.dot(p, p1b_ref[...], preferred_element_type=jnp.float32), 0.0)
    lg = jnp.dot(lat, p2b_ref[...], preferred_element_type=jnp.float32)
    lm = jnp.max(lg, axis=-1, keepdims=True)
    e2 = jnp.where(blk, jnp.exp(lg - lm), 0.0)
    a2 = e2 / jnp.sum(e2, axis=-1, keepdims=True)
    for t in range(GB // SUB):
        rows = slice(t * ROWS, (t + 1) * ROWS)
        o = jnp.dot(a2[rows], v[rows], preferred_element_type=jnp.float32)
        out_ref[t * SUB:(t + 1) * SUB] = o.reshape(SUB, BLOCK, 64)


def _sparse_kernel(x_ref, m_ref, wv_ref, sc_ref, g_ref, out_ref):
    kk = sc_ref[0, 1]
    amr = sc_ref[0, 2]
    gamma = g_ref[0, 0]
    nr = GB_S * BLOCK
    X = x_ref[...].reshape(nr, x_ref.shape[-1])
    y = jax.lax.dot_general(X, m_ref[...], _DNN, preferred_element_type=jnp.float32)
    v = jax.lax.dot_general(X, wv_ref[...], _DNT, preferred_element_type=jnp.float32)
    blk, dscale = _masks_tall(nr)
    s = _scores_all(y, X)
    f = jnp.maximum(s, 0.0) * dscale
    rb = (jax.lax.broadcasted_iota(jnp.int32, (nr, BLOCK), 0) // BLOCK) % SUB
    fc = jnp.zeros((nr, BLOCK), jnp.float32)
    for u in range(SUB):
        fc = fc + jnp.where(rb == u, f[:, u * BLOCK:(u + 1) * BLOCK], 0.0)
    mean = jnp.mean(fc, axis=-1, keepdims=True)
    var = jnp.sum((fc - mean) ** 2, axis=-1, keepdims=True) / (BLOCK - 1)
    sigma = jnp.sqrt(var)
    m = jnp.max(fc, axis=-1, keepdims=True)
    denom = jnp.maximum(m, sigma) + 1e-6
    nw = jnp.clip(jnp.floor(amr * fc / denom), 0.0, amr)
    col = jax.lax.broadcasted_iota(jnp.int32, (nr, BLOCK), 1)
    rank = jnp.zeros((nr, BLOCK), jnp.float32)
    for sft in range(1, BLOCK):
        fs = jnp.concatenate([fc[:, sft:], fc[:, :sft]], axis=1)
        cond = (fs > fc) | ((fs == fc) & (col + sft >= BLOCK))
        rank = rank + cond.astype(jnp.float32)
    sel = rank < jnp.minimum(kk, float(MAX_K))
    w = jnp.where(sel, nw, 0.0) / gamma
    wb = jnp.where(blk, jnp.concatenate([w] * SUB, axis=1), 0.0)
    for t in range(GB_S // SUB):
        rows = slice(t * ROWS, (t + 1) * ROWS)
        o = jnp.dot(wb[rows], v[rows], preferred_element_type=jnp.float32)
        out_ref[t * SUB:(t + 1) * SUB] = o.reshape(SUB, BLOCK, 64)


def kernel(x, Wk, Wq, Wv, P1, P2, gamma):
    B, T, C = x.shape
    nsteps = B // GB
    eye = jnp.eye(SUB, dtype=jnp.float32)
    p1b = jnp.kron(eye, P1.T)
    p2b = jnp.kron(eye, P2.T)
    M = jnp.dot(Wq.T, Wk, preferred_element_type=jnp.float32)

    m_spec = pl.BlockSpec((C, C), lambda i: (0, 0))
    wv_spec = pl.BlockSpec((64, C), lambda i: (0, 0))
    x_spec = pl.BlockSpec((GB, T, C), lambda i: (i, 0, 0))
    out_spec = pl.BlockSpec((GB, T, 64), lambda i: (i, 0, 0))

    out_h, sc = pl.pallas_call(
        _fused_kernel,
        grid=(nsteps,),
        in_specs=[x_spec, m_spec, wv_spec,
                  pl.BlockSpec((ROWS, ROWS), lambda i: (0, 0)),
                  pl.BlockSpec((ROWS, ROWS), lambda i: (0, 0))],
        out_specs=(out_spec, pl.BlockSpec(memory_space=pltpu.SMEM)),
        out_shape=(jax.ShapeDtypeStruct((B, T, 64), jnp.float32),
                   jax.ShapeDtypeStruct((1, 3), jnp.float32)),
        scratch_shapes=[pltpu.SMEM((1, 1), jnp.float32)],
    )(x, M, Wv, p1b, p2b)

    hybrid = sc[0, 0] > 0.5

    def _hybrid_branch(ops):
        return ops[4]

    def _sparse_branch(ops):
        x, M, Wv, sc, _, g = ops
        return pl.pallas_call(
            _sparse_kernel,
            grid=(B // GB_S,),
            in_specs=[pl.BlockSpec((GB_S, T, C), lambda i: (i, 0, 0)),
                      m_spec, wv_spec,
                      pl.BlockSpec(memory_space=pltpu.SMEM),
                      pl.BlockSpec(memory_space=pltpu.SMEM)],
            out_specs=pl.BlockSpec((GB_S, T, 64), lambda i: (i, 0, 0)),
            out_shape=jax.ShapeDtypeStruct((B, T, 64), jnp.float32),
        )(x, M, Wv, sc, g)

    g = gamma.astype(jnp.float32).reshape(1, 1)
    ops = (x, M, Wv, sc, out_h, g)
    return jax.lax.cond(hybrid, _hybrid_branch, _sparse_branch, ops)


# speedup vs baseline: 2.0596x; 1.0298x over previous
import functools
import math

import jax
import jax.numpy as jnp
from jax.experimental import pallas as pl
from jax.experimental.pallas import tpu as pltpu

BLOCK = 32
MIN_K, MAX_K, ALPHA, THR = 4, 16, 0.1, 0.5
SUB = 4
ROWS = SUB * BLOCK
GB = 128
GB_S = 16
INV_SQRT_D = 1.0 / math.sqrt(64.0)
NOB = float(ROWS - BLOCK)

_DNT = (((1,), (1,)), ((), ()))
_DNN = (((1,), (0,)), ((), ()))


def _masks_tall(nrows):
    i = jax.lax.broadcasted_iota(jnp.int32, (nrows, ROWS), 0)
    j = jax.lax.broadcasted_iota(jnp.int32, (nrows, ROWS), 1)
    blk = ((i // BLOCK) % SUB) == (j // BLOCK)
    tril = (i % BLOCK) >= (j % BLOCK)
    decay = 1.0 - ALPHA * jnp.abs(i % BLOCK - j % BLOCK).astype(jnp.float32) / BLOCK
    dscale = jnp.where(blk & tril, decay * INV_SQRT_D, 0.0)
    return blk, dscale


def _scores_all(y, xt):
    nt = y.shape[0] // ROWS
    outs = []
    for t in range(nt):
        rows = slice(t * ROWS, (t + 1) * ROWS)
        outs.append(jax.lax.dot_general(y[rows], xt[rows], _DNT,
                                        preferred_element_type=jnp.float32))
    return jnp.concatenate(outs, axis=0)


def _probs_corr(s, dscale):
    f = jnp.maximum(s, 0.0) * dscale
    m = jnp.max(f, axis=-1, keepdims=True)
    eb = jnp.exp(f - m)
    em = jnp.exp(-m)
    denom = jnp.sum(eb, axis=-1, keepdims=True) - NOB * em
    p = eb / denom
    return f, p, em / denom


def _fused_kernel(x_ref, m_ref, wv_ref, p1b_ref, p2b_ref,
                  out_ref, sc_ref, acc_ref):
    step = pl.program_id(0)
    nsteps = pl.num_programs(0)
    nr = GB * BLOCK
    X = x_ref[...].reshape(nr, x_ref.shape[-1])
    y = jax.lax.dot_general(X, m_ref[...], _DNN, preferred_element_type=jnp.float32)
    v = jax.lax.dot_general(X, wv_ref[...], _DNT, preferred_element_type=jnp.float32)
    blk, dscale = _masks_tall(nr)
    s = _scores_all(y, X)
    _, p, po = _probs_corr(s, dscale)

    full = jnp.sum(p * jnp.log(p + 1e-9))
    corr = NOB * jnp.sum(po * jnp.log(po + 1e-9))

    @pl.when(step == 0)
    def _init():
        acc_ref[0, 0] = 0.0

    acc_ref[0, 0] += corr - full

    @pl.when(step == nsteps - 1)
    def _fin():
        total_rows = float(nsteps * GB * BLOCK)
        a = acc_ref[0, 0] / (total_rows * math.log(BLOCK))
        sc_ref[0, 0] = jnp.where(a > THR, 1.0, 0.0)
        sc_ref[0, 1] = jnp.clip(jnp.floor(MIN_K + (MAX_K - MIN_K) * a),
                                float(MIN_K), float(MAX_K))
        sc_ref[0, 2] = jnp.floor(15.0 + (127.0 - 15.0) * a)

    lat = jnp.maximum(jnp.dot(p, p1b_ref[...], preferred_element_type=jnp.float32), 0.0)
    lg = jnp.dot(lat, p2b_ref[...], preferred_element_type=jnp.float32)
    lm = jnp.max(lg, axis=-1, keepdims=True)
    e2 = jnp.where(blk, jnp.exp(lg - lm), 0.0)
    a2 = e2 / jnp.sum(e2, axis=-1, keepdims=True)
    for t in range(GB // SUB):
        rows = slice(t * ROWS, (t + 1) * ROWS)
        o = jnp.dot(a2[rows], v[rows], preferred_element_type=jnp.float32)
        out_ref[t * SUB:(t + 1) * SUB] = o.reshape(SUB, BLOCK, 64)


def _sparse_kernel(x_ref, m_ref, wv_ref, sc_ref, g_ref, out_ref):
    kk = sc_ref[0, 1]
    amr = sc_ref[0, 2]
    gamma = g_ref[0, 0]
    nr = GB_S * BLOCK
    X = x_ref[...].reshape(nr, x_ref.shape[-1])
    y = jax.lax.dot_general(X, m_ref[...], _DNN, preferred_element_type=jnp.float32)
    v = jax.lax.dot_general(X, wv_ref[...], _DNT, preferred_element_type=jnp.float32)
    blk, dscale = _masks_tall(nr)
    s = _scores_all(y, X)
    f = jnp.maximum(s, 0.0) * dscale
    rb = (jax.lax.broadcasted_iota(jnp.int32, (nr, BLOCK), 0) // BLOCK) % SUB
    fc = jnp.zeros((nr, BLOCK), jnp.float32)
    for u in range(SUB):
        fc = fc + jnp.where(rb == u, f[:, u * BLOCK:(u + 1) * BLOCK], 0.0)
    mean = jnp.mean(fc, axis=-1, keepdims=True)
    var = jnp.sum((fc - mean) ** 2, axis=-1, keepdims=True) / (BLOCK - 1)
    sigma = jnp.sqrt(var)
    m = jnp.max(fc, axis=-1, keepdims=True)
    denom = jnp.maximum(m, sigma) + 1e-6
    nw = jnp.clip(jnp.floor(amr * fc / denom), 0.0, amr)
    col = jax.lax.broadcasted_iota(jnp.int32, (nr, BLOCK), 1)
    rank = jnp.zeros((nr, BLOCK), jnp.float32)
    for sft in range(1, BLOCK):
        fs = jnp.concatenate([fc[:, sft:], fc[:, :sft]], axis=1)
        cond = (fs > fc) | ((fs == fc) & (col + sft >= BLOCK))
        rank = rank + cond.astype(jnp.float32)
    sel = rank < jnp.minimum(kk, float(MAX_K))
    w = jnp.where(sel, nw, 0.0) / gamma
    wb = jnp.where(blk, jnp.concatenate([w] * SUB, axis=1), 0.0)
    for t in range(GB_S // SUB):
        rows = slice(t * ROWS, (t + 1) * ROWS)
        o = jnp.dot(wb[rows], v[rows], preferred_element_type=jnp.float32)
        out_ref[t * SUB:(t + 1) * SUB] = o.reshape(SUB, BLOCK, 64)


def kernel(x, Wk, Wq, Wv, P1, P2, gamma):
    B, T, C = x.shape
    nsteps = B // GB
    eye = jnp.eye(SUB, dtype=jnp.float32)
    p1b = jnp.kron(eye, P1.T)
    p2b = jnp.kron(eye, P2.T)
    M = jnp.dot(Wq.T, Wk, preferred_element_type=jnp.float32)

    m_spec = pl.BlockSpec((C, C), lambda i: (0, 0))
    wv_spec = pl.BlockSpec((64, C), lambda i: (0, 0))
    x_spec = pl.BlockSpec((GB, T, C), lambda i: (i, 0, 0))
    out_spec = pl.BlockSpec((GB, T, 64), lambda i: (i, 0, 0))

    out_h, sc = pl.pallas_call(
        _fused_kernel,
        grid=(nsteps,),
        in_specs=[x_spec, m_spec, wv_spec,
                  pl.BlockSpec((ROWS, ROWS), lambda i: (0, 0)),
                  pl.BlockSpec((ROWS, ROWS), lambda i: (0, 0))],
        out_specs=(out_spec, pl.BlockSpec(memory_space=pltpu.SMEM)),
        out_shape=(jax.ShapeDtypeStruct((B, T, 64), jnp.float32),
                   jax.ShapeDtypeStruct((1, 3), jnp.float32)),
        scratch_shapes=[pltpu.SMEM((1, 1), jnp.float32)],
    )(x, M, Wv, p1b, p2b)

    hybrid = sc[0, 0] > 0.5

    def _hybrid_branch(ops):
        return ops[4]

    def _sparse_branch(ops):
        x, M, Wv, sc, _, g = ops
        return pl.pallas_call(
            _sparse_kernel,
            grid=(B // GB_S,),
            in_specs=[pl.BlockSpec((GB_S, T, C), lambda i: (i, 0, 0)),
                      m_spec, wv_spec,
                      pl.BlockSpec(memory_space=pltpu.SMEM),
                      pl.BlockSpec(memory_space=pltpu.SMEM)],
            out_specs=pl.BlockSpec((GB_S, T, 64), lambda i: (i, 0, 0)),
            out_shape=jax.ShapeDtypeStruct((B, T, 64), jnp.float32),
        )(x, M, Wv, sc, g)

    g = gamma.astype(jnp.float32).reshape(1, 1)
    ops = (x, M, Wv, sc, out_h, g)
    return jax.lax.cond(hybrid, _hybrid_branch, _sparse_branch, ops)
